# baseline (speedup 1.0000x reference)
import functools

import jax
import jax.numpy as jnp
from jax import lax
from jax.experimental import pallas as pl
from jax.experimental.pallas import tpu as pltpu
from jax.experimental.pallas import tpu_sc as plsc

B = 16384
L = 20
D = 128
D_OUT = 2 * D

NUM_WORKERS = 32
ROWS_PER_W = B // NUM_WORKERS
CHUNK = 16
N_CHUNKS = ROWS_PER_W // CHUNK
LANES = 16
CL = CHUNK * L
UNROLL = 2


def _body(titles_hbm, tokens_hbm, title_tab, text_tab, out_hbm,
          tokbuf0, tokbuf1, tidx0, tidx1, tokrows0, tokrows1,
          trows0, trows1, outbuf0, outbuf1, row0buf, nzbuf, recbuf,
          sem_tok0, sem_tok1, sem_ttl0, sem_ttl1):
    tokbuf = (tokbuf0, tokbuf1)
    tidx = (tidx0, tidx1)
    tokrows = (tokrows0, tokrows1)
    trows = (trows0, trows1)
    outbuf = (outbuf0, outbuf1)
    sem_tok = (sem_tok0, sem_tok1)
    sem_ttl = (sem_ttl0, sem_ttl1)

    wid = lax.axis_index("s") * 2 + lax.axis_index("c")
    base = wid * ROWS_PER_W
    iota = lax.iota(jnp.int32, LANES)

    pltpu.sync_copy(text_tab.at[pl.ds(0, 1)], row0buf)
    row0v = [row0buf[0, pl.ds(j * LANES, LANES)] for j in range(D // LANES)]

    def prefetch(chunk, p):
        row0 = base + chunk * CHUNK
        pltpu.sync_copy(tokens_hbm.at[pl.ds(row0 * L, CL)], tokbuf[p])
        pltpu.async_copy(text_tab.at[tokbuf[p]], tokrows[p], sem_tok[p])
        pltpu.sync_copy(titles_hbm.at[pl.ds(row0, CHUNK)], tidx[p])
        pltpu.async_copy(title_tab.at[tidx[p]], trows[p], sem_ttl[p])

    def compute(chunk, p):
        row0 = base + chunk * CHUNK

        nz = jnp.zeros((LANES,), jnp.float32)
        for t in range(L):
            tv = plsc.load_gather(tokbuf[p], [iota * L + t])
            nz = nz + jnp.where(tv == 0, 1.0, 0.0)
        nzbuf[...] = nz
        recbuf[...] = 1.0 / jnp.maximum(jnp.float32(L) - nz, 1.0)

        pltpu.make_async_copy(text_tab.at[tokbuf[p]], tokrows[p],
                              sem_tok[p]).wait()
        pltpu.make_async_copy(title_tab.at[tidx[p]], trows[p],
                              sem_ttl[p]).wait()

        def row_body(i, _):
            for u in range(UNROLL):
                r = i * UNROLL + u
                splat = jnp.full((LANES,), r, jnp.int32)
                nzv = plsc.load_gather(nzbuf, [splat])
                rec = plsc.load_gather(recbuf, [splat])
                for j in range(D // LANES):
                    s = tokrows[p][r * L, pl.ds(j * LANES, LANES)]
                    for t in range(1, L):
                        s = s + tokrows[p][r * L + t, pl.ds(j * LANES, LANES)]
                    outbuf[p][r, pl.ds(j * LANES, LANES)] = \
                        trows[p][r, pl.ds(j * LANES, LANES)]
                    outbuf[p][r, pl.ds(D + j * LANES, LANES)] = \
                        (s - nzv * row0v[j]) * rec
            return 0

        lax.fori_loop(0, CHUNK // UNROLL, row_body, 0)
        pltpu.sync_copy(outbuf[p], out_hbm.at[pl.ds(row0, CHUNK)])

    prefetch(0, 0)

    def outer(i, _):
        for p in range(2):
            chunk = i * 2 + p

            @pl.when(chunk + 1 < N_CHUNKS)
            def _():
                prefetch(chunk + 1, 1 - p)

            compute(chunk, p)
        return 0

    lax.fori_loop(0, N_CHUNKS // 2, outer, 0)


@functools.partial(jax.jit, static_argnums=())
def _sc_call(titles_i, tokens_i, title_table, text_table):
    mesh = plsc.VectorSubcoreMesh(core_axis_name="c", subcore_axis_name="s")
    return pl.kernel(
        _body,
        out_type=jax.ShapeDtypeStruct((B, D_OUT), jnp.float32),
        mesh=mesh,
        scratch_types=[
            pltpu.VMEM((CL,), jnp.int32),
            pltpu.VMEM((CL,), jnp.int32),
            pltpu.VMEM((CHUNK,), jnp.int32),
            pltpu.VMEM((CHUNK,), jnp.int32),
            pltpu.VMEM((CL, D), jnp.float32),
            pltpu.VMEM((CL, D), jnp.float32),
            pltpu.VMEM((CHUNK, D), jnp.float32),
            pltpu.VMEM((CHUNK, D), jnp.float32),
            pltpu.VMEM((CHUNK, D_OUT), jnp.float32),
            pltpu.VMEM((CHUNK, D_OUT), jnp.float32),
            pltpu.VMEM((1, D), jnp.float32),
            pltpu.VMEM((LANES,), jnp.float32),
            pltpu.VMEM((LANES,), jnp.float32),
            pltpu.SemaphoreType.DMA,
            pltpu.SemaphoreType.DMA,
            pltpu.SemaphoreType.DMA,
            pltpu.SemaphoreType.DMA,
        ],
        compiler_params=pltpu.CompilerParams(needs_layout_passes=False),
    )(titles_i, tokens_i, title_table, text_table)


def kernel(titles, tokens, title_table, text_table):
    titles_i = titles.astype(jnp.int32)
    tokens_i = tokens.reshape(-1).astype(jnp.int32)
    return _sc_call(titles_i, tokens_i, title_table, text_table)


# speedup vs baseline: 1.1027x; 1.1027x over previous
import functools

import jax
import jax.numpy as jnp
from jax import lax
from jax.experimental import pallas as pl
from jax.experimental.pallas import tpu as pltpu
from jax.experimental.pallas import tpu_sc as plsc

B = 16384
L = 20
D = 128
D_OUT = 2 * D

NUM_WORKERS = 32
ROWS_PER_W = B // NUM_WORKERS
CHUNK = 16
N_CHUNKS = ROWS_PER_W // CHUNK
LANES = 16
CL = CHUNK * L
UNROLL = 2


def _body(titles_hbm, tokens_hbm, title_tab, text_tab, out_hbm,
          tokbuf0, tokbuf1, tidx0, tidx1, tokrows0, tokrows1,
          outbuf0, outbuf1, row0buf, nzbuf, recbuf,
          sem_tok0, sem_tok1, sem_ttl0, sem_ttl1,
          sem_idx0, sem_idx1, sem_out0, sem_out1):
    tokbuf = (tokbuf0, tokbuf1)
    tidx = (tidx0, tidx1)
    tokrows = (tokrows0, tokrows1)
    outbuf = (outbuf0, outbuf1)
    sem_tok = (sem_tok0, sem_tok1)
    sem_ttl = (sem_ttl0, sem_ttl1)
    sem_idx = (sem_idx0, sem_idx1)
    sem_out = (sem_out0, sem_out1)

    wid = lax.axis_index("s") * 2 + lax.axis_index("c")
    base = wid * ROWS_PER_W
    iota = lax.iota(jnp.int32, LANES)

    pltpu.sync_copy(text_tab.at[pl.ds(0, 1)], row0buf)
    row0v = [row0buf[0, pl.ds(j * LANES, LANES)] for j in range(D // LANES)]

    def idx_copy(chunk, p):
        row0 = base + chunk * CHUNK
        pltpu.async_copy(tokens_hbm.at[pl.ds(row0 * L, CL)], tokbuf[p],
                         sem_idx[p])
        pltpu.async_copy(titles_hbm.at[pl.ds(row0, CHUNK)], tidx[p],
                         sem_idx[p])

    def idx_wait(chunk, p):
        row0 = base + chunk * CHUNK
        pltpu.make_async_copy(tokens_hbm.at[pl.ds(row0 * L, CL)], tokbuf[p],
                              sem_idx[p]).wait()
        pltpu.make_async_copy(titles_hbm.at[pl.ds(row0, CHUNK)], tidx[p],
                              sem_idx[p]).wait()

    def gather_issue(p):
        pltpu.async_copy(text_tab.at[tokbuf[p]], tokrows[p], sem_tok[p])
        pltpu.async_copy(title_tab.at[tidx[p]],
                         outbuf[p].at[:, pl.ds(0, D)], sem_ttl[p])

    def gather_wait(p):
        pltpu.make_async_copy(text_tab.at[tokbuf[p]], tokrows[p],
                              sem_tok[p]).wait()
        pltpu.make_async_copy(title_tab.at[tidx[p]],
                              outbuf[p].at[:, pl.ds(0, D)], sem_ttl[p]).wait()

    def out_issue(chunk, p):
        row0 = base + chunk * CHUNK
        pltpu.async_copy(outbuf[p], out_hbm.at[pl.ds(row0, CHUNK)],
                         sem_out[p])

    def out_wait(chunk, p):
        row0 = base + chunk * CHUNK
        pltpu.make_async_copy(outbuf[p], out_hbm.at[pl.ds(row0, CHUNK)],
                              sem_out[p]).wait()

    def compute_rows(p):
        nz = jnp.zeros((LANES,), jnp.float32)
        for t in range(L):
            tv = plsc.load_gather(tokbuf[p], [iota * L + t])
            nz = nz + jnp.where(tv == 0, 1.0, 0.0)
        nzbuf[pl.ds(0, LANES)] = nz
        recbuf[pl.ds(0, LANES)] = 1.0 / jnp.maximum(jnp.float32(L) - nz, 1.0)

        def row_body(i, _):
            for u in range(UNROLL):
                r = i * UNROLL + u
                nzv = jnp.full((LANES,), nzbuf[pl.ds(r, LANES)][0],
                               jnp.float32)
                rec = jnp.full((LANES,), recbuf[pl.ds(r, LANES)][0],
                               jnp.float32)
                for j in range(D // LANES):
                    s = tokrows[p][r * L, pl.ds(j * LANES, LANES)]
                    for t in range(1, L):
                        s = s + tokrows[p][r * L + t, pl.ds(j * LANES, LANES)]
                    outbuf[p][r, pl.ds(D + j * LANES, LANES)] = \
                        (s - nzv * row0v[j]) * rec
            return 0

        lax.fori_loop(0, CHUNK // UNROLL, row_body, 0)

    row00 = base
    pltpu.sync_copy(tokens_hbm.at[pl.ds(row00 * L, CL)], tokbuf[0])
    pltpu.sync_copy(titles_hbm.at[pl.ds(row00, CHUNK)], tidx[0])
    gather_issue(0)
    idx_copy(1, 1)

    def outer(i, _):
        for p in range(2):
            chunk = i * 2 + p
            gather_wait(p)

            @pl.when(chunk + 1 < N_CHUNKS)
            def _():
                idx_wait(chunk + 1, 1 - p)

                @pl.when(chunk >= 1)
                def _():
                    out_wait(chunk - 1, 1 - p)

                gather_issue(1 - p)

            compute_rows(p)

            @pl.when(chunk + 2 < N_CHUNKS)
            def _():
                idx_copy(chunk + 2, p)

            out_issue(chunk, p)
        return 0

    lax.fori_loop(0, N_CHUNKS // 2, outer, 0)
    out_wait(N_CHUNKS - 2, (N_CHUNKS - 2) % 2)
    out_wait(N_CHUNKS - 1, (N_CHUNKS - 1) % 2)


@functools.partial(jax.jit, static_argnums=())
def _sc_call(titles_i, tokens_i, title_table, text_table):
    mesh = plsc.VectorSubcoreMesh(core_axis_name="c", subcore_axis_name="s")
    return pl.kernel(
        _body,
        out_type=jax.ShapeDtypeStruct((B, D_OUT), jnp.float32),
        mesh=mesh,
        scratch_types=[
            pltpu.VMEM((CL,), jnp.int32),
            pltpu.VMEM((CL,), jnp.int32),
            pltpu.VMEM((CHUNK,), jnp.int32),
            pltpu.VMEM((CHUNK,), jnp.int32),
            pltpu.VMEM((CL, D), jnp.float32),
            pltpu.VMEM((CL, D), jnp.float32),
            pltpu.VMEM((CHUNK, D_OUT), jnp.float32),
            pltpu.VMEM((CHUNK, D_OUT), jnp.float32),
            pltpu.VMEM((1, D), jnp.float32),
            pltpu.VMEM((2 * LANES,), jnp.float32),
            pltpu.VMEM((2 * LANES,), jnp.float32),
            pltpu.SemaphoreType.DMA,
            pltpu.SemaphoreType.DMA,
            pltpu.SemaphoreType.DMA,
            pltpu.SemaphoreType.DMA,
            pltpu.SemaphoreType.DMA,
            pltpu.SemaphoreType.DMA,
            pltpu.SemaphoreType.DMA,
            pltpu.SemaphoreType.DMA,
        ],
        compiler_params=pltpu.CompilerParams(needs_layout_passes=False),
    )(titles_i, tokens_i, title_table, text_table)


def kernel(titles, tokens, title_table, text_table):
    titles_i = titles.astype(jnp.int32)
    tokens_i = tokens.reshape(-1).astype(jnp.int32)
    return _sc_call(titles_i, tokens_i, title_table, text_table)


# speedup vs baseline: 1.2555x; 1.1386x over previous
import functools

import jax
import jax.numpy as jnp
from jax import lax
from jax.experimental import pallas as pl
from jax.experimental.pallas import tpu as pltpu
from jax.experimental.pallas import tpu_sc as plsc

B = 16384
L = 20
D = 128
D_OUT = 2 * D

NUM_WORKERS = 32
ROWS_PER_W = B // NUM_WORKERS
CHUNK = 16
N_CHUNKS = ROWS_PER_W // CHUNK
LANES = 16
CL = CHUNK * L
NOUT = 4


def _body(titles_hbm, tokens_hbm, title_tab, text_tab, out_hbm,
          tokbuf0, tokbuf1, tidx0, tidx1, tokrows0, tokrows1,
          outbuf0, outbuf1, outbuf2, outbuf3, acc0, acc1, accv, zerobuf,
          segbuf, row0buf, nzbuf0, nzbuf1, recbuf0, recbuf1,
          sem_tok0, sem_tok1, sem_ttl0, sem_ttl1,
          sem_idx0, sem_idx1, sem_acc0, sem_acc1, sem_zero0, sem_zero1,
          sem_out0, sem_out1, sem_out2, sem_out3):
    tokbuf = (tokbuf0, tokbuf1)
    tidx = (tidx0, tidx1)
    tokrows = (tokrows0, tokrows1)
    outbuf = (outbuf0, outbuf1, outbuf2, outbuf3)
    acc = (acc0, acc1)
    nzbuf = (nzbuf0, nzbuf1)
    recbuf = (recbuf0, recbuf1)
    sem_tok = (sem_tok0, sem_tok1)
    sem_ttl = (sem_ttl0, sem_ttl1)
    sem_idx = (sem_idx0, sem_idx1)
    sem_acc = (sem_acc0, sem_acc1)
    sem_zero = (sem_zero0, sem_zero1)
    sem_out = (sem_out0, sem_out1, sem_out2, sem_out3)

    wid = lax.axis_index("s") * 2 + lax.axis_index("c")
    sid = lax.axis_index("s")
    base = wid * ROWS_PER_W
    iota = lax.iota(jnp.int32, LANES)
    zv = jnp.zeros((LANES,), jnp.float32)

    pltpu.sync_copy(text_tab.at[pl.ds(0, 1)], row0buf)
    row0v = [row0buf[0, pl.ds(j * LANES, LANES)] for j in range(D // LANES)]
    for r in range(CHUNK):
        for j in range(D // LANES):
            zerobuf[r, pl.ds(j * LANES, LANES)] = zv
    for k in range(CL // LANES):
        segbuf[pl.ds(k * LANES, LANES)] = \
            sid * CHUNK + (iota + k * LANES) // L

    def idx_copy(chunk, p):
        row0 = base + chunk * CHUNK
        pltpu.async_copy(tokens_hbm.at[pl.ds(row0 * L, CL)], tokbuf[p],
                         sem_idx[p])
        pltpu.async_copy(titles_hbm.at[pl.ds(row0, CHUNK)], tidx[p],
                         sem_idx[p])

    def idx_wait(chunk, p):
        row0 = base + chunk * CHUNK
        pltpu.make_async_copy(tokens_hbm.at[pl.ds(row0 * L, CL)], tokbuf[p],
                              sem_idx[p]).wait()
        pltpu.make_async_copy(titles_hbm.at[pl.ds(row0, CHUNK)], tidx[p],
                              sem_idx[p]).wait()

    def gather_issue(p, o):
        pltpu.async_copy(text_tab.at[tokbuf[p]], tokrows[p], sem_tok[p])
        pltpu.async_copy(title_tab.at[tidx[p]],
                         outbuf[o].at[:, pl.ds(0, D)], sem_ttl[p])

    def gather_wait(p, o):
        pltpu.make_async_copy(text_tab.at[tokbuf[p]], tokrows[p],
                              sem_tok[p]).wait()
        pltpu.make_async_copy(title_tab.at[tidx[p]],
                              outbuf[o].at[:, pl.ds(0, D)], sem_ttl[p]).wait()

    def out_issue(chunk, o):
        row0 = base + chunk * CHUNK
        pltpu.async_copy(outbuf[o], out_hbm.at[pl.ds(row0, CHUNK)],
                         sem_out[o])

    def out_wait(chunk, o):
        row0 = base + chunk * CHUNK
        pltpu.make_async_copy(outbuf[o], out_hbm.at[pl.ds(row0, CHUNK)],
                              sem_out[o]).wait()

    def zero_issue(p):
        pltpu.async_copy(zerobuf, acc[p].at[pl.ds(sid * CHUNK, CHUNK)],
                         sem_zero[p])

    def zero_wait(p):
        pltpu.make_async_copy(zerobuf, acc[p].at[pl.ds(sid * CHUNK, CHUNK)],
                              sem_zero[p]).wait()

    def counts(p):
        nz = jnp.zeros((LANES,), jnp.float32)
        for t in range(L):
            tv = plsc.load_gather(tokbuf[p], [iota * L + t])
            nz = nz + jnp.where(tv == 0, 1.0, 0.0)
        nzbuf[p][pl.ds(0, LANES)] = nz
        recbuf[p][pl.ds(0, LANES)] = \
            1.0 / jnp.maximum(jnp.float32(L) - nz, 1.0)

    def fixup(p, o):
        pltpu.sync_copy(acc[p].at[pl.ds(sid * CHUNK, CHUNK)], accv)
        for r in range(CHUNK):
            nzv = jnp.full((LANES,), nzbuf[p][pl.ds(r, LANES)][0],
                           jnp.float32)
            rec = jnp.full((LANES,), recbuf[p][pl.ds(r, LANES)][0],
                           jnp.float32)
            for j in range(D // LANES):
                a = accv[r, pl.ds(j * LANES, LANES)]
                outbuf[o][r, pl.ds(D + j * LANES, LANES)] = \
                    (a - nzv * row0v[j]) * rec

    pltpu.sync_copy(tokens_hbm.at[pl.ds(base * L, CL)], tokbuf[0])
    pltpu.sync_copy(titles_hbm.at[pl.ds(base, CHUNK)], tidx[0])
    gather_issue(0, 0)
    zero_issue(0)
    idx_copy(1, 1)

    def outer(i, _):
        for u in range(NOUT):
            c = i * NOUT + u
            p = u % 2
            o = u
            pprev = 1 - p
            oprev = (u - 1) % NOUT

            gather_wait(p, o)

            @pl.when(c >= 1)
            def _():
                pltpu.make_async_copy(tokrows[pprev],
                                      acc[pprev].at[segbuf],
                                      sem_acc[pprev]).wait()
                fixup(pprev, oprev)
                out_issue(c - 1, oprev)

            @pl.when(c + 1 < N_CHUNKS)
            def _():
                zero_issue(pprev)

            zero_wait(p)
            pltpu.async_copy(tokrows[p], acc[p].at[segbuf], sem_acc[p],
                             add=True)
            counts(p)

            @pl.when(c + 1 < N_CHUNKS)
            def _():
                idx_wait(c + 1, pprev)

                @pl.when(c >= 3)
                def _():
                    out_wait(c - 3, (u + 1) % NOUT)

                gather_issue(pprev, (u + 1) % NOUT)

            @pl.when(c + 2 < N_CHUNKS)
            def _():
                idx_copy(c + 2, p)
        return 0

    lax.fori_loop(0, N_CHUNKS // NOUT, outer, 0)

    lastp = (N_CHUNKS - 1) % 2
    lasto = (N_CHUNKS - 1) % NOUT
    pltpu.make_async_copy(tokrows[lastp], acc[lastp].at[segbuf],
                          sem_acc[lastp]).wait()
    fixup(lastp, lasto)
    out_issue(N_CHUNKS - 1, lasto)
    for k in range(NOUT):
        c = N_CHUNKS - NOUT + k
        out_wait(c, c % NOUT)


@functools.partial(jax.jit, static_argnums=())
def _sc_call(titles_i, tokens_i, title_table, text_table):
    mesh = plsc.VectorSubcoreMesh(core_axis_name="c", subcore_axis_name="s")
    return pl.kernel(
        _body,
        out_type=jax.ShapeDtypeStruct((B, D_OUT), jnp.float32),
        mesh=mesh,
        scratch_types=[
            pltpu.VMEM((CL,), jnp.int32),
            pltpu.VMEM((CL,), jnp.int32),
            pltpu.VMEM((CHUNK,), jnp.int32),
            pltpu.VMEM((CHUNK,), jnp.int32),
            pltpu.VMEM((CL, D), jnp.float32),
            pltpu.VMEM((CL, D), jnp.float32),
            pltpu.VMEM((CHUNK, D_OUT), jnp.float32),
            pltpu.VMEM((CHUNK, D_OUT), jnp.float32),
            pltpu.VMEM((CHUNK, D_OUT), jnp.float32),
            pltpu.VMEM((CHUNK, D_OUT), jnp.float32),
            pltpu.VMEM_SHARED((16 * CHUNK, D), jnp.float32),
            pltpu.VMEM_SHARED((16 * CHUNK, D), jnp.float32),
            pltpu.VMEM((CHUNK, D), jnp.float32),
            pltpu.VMEM((CHUNK, D), jnp.float32),
            pltpu.VMEM((CL,), jnp.int32),
            pltpu.VMEM((1, D), jnp.float32),
            pltpu.VMEM((2 * LANES,), jnp.float32),
            pltpu.VMEM((2 * LANES,), jnp.float32),
            pltpu.VMEM((2 * LANES,), jnp.float32),
            pltpu.VMEM((2 * LANES,), jnp.float32),
            pltpu.SemaphoreType.DMA,
            pltpu.SemaphoreType.DMA,
            pltpu.SemaphoreType.DMA,
            pltpu.SemaphoreType.DMA,
            pltpu.SemaphoreType.DMA,
            pltpu.SemaphoreType.DMA,
            pltpu.SemaphoreType.DMA,
            pltpu.SemaphoreType.DMA,
            pltpu.SemaphoreType.DMA,
            pltpu.SemaphoreType.DMA,
            pltpu.SemaphoreType.DMA,
            pltpu.SemaphoreType.DMA,
            pltpu.SemaphoreType.DMA,
            pltpu.SemaphoreType.DMA,
        ],
        compiler_params=pltpu.CompilerParams(needs_layout_passes=False),
    )(titles_i, tokens_i, title_table, text_table)


def kernel(titles, tokens, title_table, text_table):
    titles_i = titles.astype(jnp.int32)
    tokens_i = tokens.reshape(-1).astype(jnp.int32)
    return _sc_call(titles_i, tokens_i, title_table, text_table)


# speedup vs baseline: 1.3524x; 1.0771x over previous
import functools

import jax
import jax.numpy as jnp
from jax import lax
from jax.experimental import pallas as pl
from jax.experimental.pallas import tpu as pltpu
from jax.experimental.pallas import tpu_sc as plsc

B = 16384
L = 20
D = 128
D_OUT = 2 * D

NUM_WORKERS = 32
ROWS_PER_W = B // NUM_WORKERS
CHUNK = 16
N_CHUNKS = ROWS_PER_W // CHUNK
LANES = 16
CL = CHUNK * L
NOUT = 4
UNROLL = 2


def _body(titles_hbm, tokens_hbm, title_tab, text_tab, out_hbm,
          tokbuf0, tokbuf1, tidx0, tidx1, tokrows0, tokrows1,
          outbuf0, outbuf1, outbuf2, outbuf3,
          row0buf, nzbuf0, nzbuf1, recbuf0, recbuf1,
          sem_tok0, sem_tok1, sem_ttl0, sem_ttl1,
          sem_idx0, sem_idx1, sem_out0, sem_out1, sem_out2, sem_out3):
    tokbuf = (tokbuf0, tokbuf1)
    tidx = (tidx0, tidx1)
    tokrows = (tokrows0, tokrows1)
    outbuf = (outbuf0, outbuf1, outbuf2, outbuf3)
    nzbuf = (nzbuf0, nzbuf1)
    recbuf = (recbuf0, recbuf1)
    sem_tok = (sem_tok0, sem_tok1)
    sem_ttl = (sem_ttl0, sem_ttl1)
    sem_idx = (sem_idx0, sem_idx1)
    sem_out = (sem_out0, sem_out1, sem_out2, sem_out3)

    wid = lax.axis_index("s") * 2 + lax.axis_index("c")
    base = wid * ROWS_PER_W
    iota = lax.iota(jnp.int32, LANES)

    pltpu.sync_copy(text_tab.at[pl.ds(0, 1)], row0buf)
    row0v = [row0buf[0, pl.ds(j * LANES, LANES)] for j in range(D // LANES)]

    def idx_copy(chunk, p):
        row0 = base + chunk * CHUNK
        pltpu.async_copy(tokens_hbm.at[pl.ds(row0 * L, CL)], tokbuf[p],
                         sem_idx[p])
        pltpu.async_copy(titles_hbm.at[pl.ds(row0, CHUNK)], tidx[p],
                         sem_idx[p])

    def idx_wait(chunk, p):
        row0 = base + chunk * CHUNK
        pltpu.make_async_copy(tokens_hbm.at[pl.ds(row0 * L, CL)], tokbuf[p],
                              sem_idx[p]).wait()
        pltpu.make_async_copy(titles_hbm.at[pl.ds(row0, CHUNK)], tidx[p],
                              sem_idx[p]).wait()

    def gather_issue(p, o):
        pltpu.async_copy(text_tab.at[tokbuf[p]], tokrows[p], sem_tok[p])
        pltpu.async_copy(title_tab.at[tidx[p]],
                         outbuf[o].at[:, pl.ds(0, D)], sem_ttl[p])

    def gather_wait(p, o):
        pltpu.make_async_copy(text_tab.at[tokbuf[p]], tokrows[p],
                              sem_tok[p]).wait()
        pltpu.make_async_copy(title_tab.at[tidx[p]],
                              outbuf[o].at[:, pl.ds(0, D)], sem_ttl[p]).wait()

    def out_issue(chunk, o):
        row0 = base + chunk * CHUNK
        pltpu.async_copy(outbuf[o], out_hbm.at[pl.ds(row0, CHUNK)],
                         sem_out[o])

    def out_wait(chunk, o):
        row0 = base + chunk * CHUNK
        pltpu.make_async_copy(outbuf[o], out_hbm.at[pl.ds(row0, CHUNK)],
                              sem_out[o]).wait()

    def counts(p):
        nz = jnp.zeros((LANES,), jnp.float32)
        for t in range(L):
            tv = plsc.load_gather(tokbuf[p], [iota * L + t])
            nz = nz + jnp.where(tv == 0, 1.0, 0.0)
        nzbuf[p][pl.ds(0, LANES)] = nz
        recbuf[p][pl.ds(0, LANES)] = \
            1.0 / jnp.maximum(jnp.float32(L) - nz, 1.0)

    def compute(p, o):
        def row_body(i, _):
            for u in range(UNROLL):
                r = i * UNROLL + u
                nzv = jnp.full((LANES,), nzbuf[p][pl.ds(r, LANES)][0],
                               jnp.float32)
                rec = jnp.full((LANES,), recbuf[p][pl.ds(r, LANES)][0],
                               jnp.float32)
                for j in range(D // LANES):
                    vals = [tokrows[p][r * L + t, pl.ds(j * LANES, LANES)]
                            for t in range(L)]
                    while len(vals) > 1:
                        nxt = [vals[k] + vals[k + 1]
                               for k in range(0, len(vals) - 1, 2)]
                        if len(vals) % 2:
                            nxt.append(vals[-1])
                        vals = nxt
                    outbuf[o][r, pl.ds(D + j * LANES, LANES)] = \
                        (vals[0] - nzv * row0v[j]) * rec
            return 0

        lax.fori_loop(0, CHUNK // UNROLL, row_body, 0)

    pltpu.sync_copy(tokens_hbm.at[pl.ds(base * L, CL)], tokbuf[0])
    pltpu.sync_copy(titles_hbm.at[pl.ds(base, CHUNK)], tidx[0])
    counts(0)
    gather_issue(0, 0)
    idx_copy(1, 1)

    def outer(i, _):
        for u in range(NOUT):
            c = i * NOUT + u
            p = u % 2
            o = u
            pprev = 1 - p

            gather_wait(p, o)

            @pl.when(c + 1 < N_CHUNKS)
            def _():
                idx_wait(c + 1, pprev)

                @pl.when(c >= 3)
                def _():
                    out_wait(c - 3, (u + 1) % NOUT)

                counts(pprev)
                gather_issue(pprev, (u + 1) % NOUT)

            compute(p, o)
            out_issue(c, o)

            @pl.when(c + 2 < N_CHUNKS)
            def _():
                idx_copy(c + 2, p)
        return 0

    lax.fori_loop(0, N_CHUNKS // NOUT, outer, 0)

    for k in range(NOUT):
        c = N_CHUNKS - NOUT + k
        out_wait(c, c % NOUT)


@functools.partial(jax.jit, static_argnums=())
def _sc_call(titles_i, tokens_i, title_table, text_table):
    mesh = plsc.VectorSubcoreMesh(core_axis_name="c", subcore_axis_name="s")
    return pl.kernel(
        _body,
        out_type=jax.ShapeDtypeStruct((B, D_OUT), jnp.float32),
        mesh=mesh,
        scratch_types=[
            pltpu.VMEM((CL,), jnp.int32),
            pltpu.VMEM((CL,), jnp.int32),
            pltpu.VMEM((CHUNK,), jnp.int32),
            pltpu.VMEM((CHUNK,), jnp.int32),
            pltpu.VMEM((CL, D), jnp.float32),
            pltpu.VMEM((CL, D), jnp.float32),
            pltpu.VMEM((CHUNK, D_OUT), jnp.float32),
            pltpu.VMEM((CHUNK, D_OUT), jnp.float32),
            pltpu.VMEM((CHUNK, D_OUT), jnp.float32),
            pltpu.VMEM((CHUNK, D_OUT), jnp.float32),
            pltpu.VMEM((1, D), jnp.float32),
            pltpu.VMEM((2 * LANES,), jnp.float32),
            pltpu.VMEM((2 * LANES,), jnp.float32),
            pltpu.VMEM((2 * LANES,), jnp.float32),
            pltpu.VMEM((2 * LANES,), jnp.float32),
            pltpu.SemaphoreType.DMA,
            pltpu.SemaphoreType.DMA,
            pltpu.SemaphoreType.DMA,
            pltpu.SemaphoreType.DMA,
            pltpu.SemaphoreType.DMA,
            pltpu.SemaphoreType.DMA,
            pltpu.SemaphoreType.DMA,
            pltpu.SemaphoreType.DMA,
            pltpu.SemaphoreType.DMA,
            pltpu.SemaphoreType.DMA,
        ],
        compiler_params=pltpu.CompilerParams(needs_layout_passes=False),
    )(titles_i, tokens_i, title_table, text_table)


def kernel(titles, tokens, title_table, text_table):
    titles_i = titles.astype(jnp.int32)
    tokens_i = tokens.reshape(-1).astype(jnp.int32)
    return _sc_call(titles_i, tokens_i, title_table, text_table)


# speedup vs baseline: 1.5721x; 1.1625x over previous
import functools

import jax
import jax.numpy as jnp
from jax import lax
from jax.experimental import pallas as pl
from jax.experimental.pallas import tpu as pltpu
from jax.experimental.pallas import tpu_sc as plsc

B = 16384
L = 20
D = 128
D_OUT = 2 * D

NUM_WORKERS = 32
ROWS_PER_W = B // NUM_WORKERS
CHUNK = 16
N_CHUNKS = ROWS_PER_W // CHUNK
LANES = 16
CL = CHUNK * L
NOUT = 4
UNROLL = 2


def _body(titles_hbm, tokens_hbm, title_tab, text_tab, out_hbm,
          tokbuf0, tokbuf1, tidx0, tidx1, tokrows0, tokrows1,
          outbuf0, outbuf1, outbuf2, outbuf3,
          row0buf, nzbuf0, nzbuf1, recbuf0, recbuf1,
          sem_tok0, sem_tok1, sem_ttl0, sem_ttl1,
          sem_idx0, sem_idx1, sem_out0, sem_out1, sem_out2, sem_out3):
    tokbuf = (tokbuf0, tokbuf1)
    tidx = (tidx0, tidx1)
    tokrows = (tokrows0, tokrows1)
    outbuf = (outbuf0, outbuf1, outbuf2, outbuf3)
    nzbuf = (nzbuf0, nzbuf1)
    recbuf = (recbuf0, recbuf1)
    sem_tok = (sem_tok0, sem_tok1)
    sem_ttl = (sem_ttl0, sem_ttl1)
    sem_idx = (sem_idx0, sem_idx1)
    sem_out = (sem_out0, sem_out1, sem_out2, sem_out3)

    wid = lax.axis_index("s") * 2 + lax.axis_index("c")
    base = wid * ROWS_PER_W
    iota = lax.iota(jnp.int32, LANES)

    pltpu.sync_copy(text_tab.at[pl.ds(0, 1)], row0buf)
    row0v = [row0buf[0, pl.ds(j * LANES, LANES)] for j in range(D // LANES)]

    def idx_copy(chunk, p):
        row0 = base + chunk * CHUNK
        pltpu.async_copy(tokens_hbm.at[pl.ds(row0 * L, CL)], tokbuf[p],
                         sem_idx[p])
        pltpu.async_copy(titles_hbm.at[pl.ds(row0, CHUNK)], tidx[p],
                         sem_idx[p])

    def idx_wait(chunk, p):
        row0 = base + chunk * CHUNK
        pltpu.make_async_copy(tokens_hbm.at[pl.ds(row0 * L, CL)], tokbuf[p],
                              sem_idx[p]).wait()
        pltpu.make_async_copy(titles_hbm.at[pl.ds(row0, CHUNK)], tidx[p],
                              sem_idx[p]).wait()

    def gather_issue(p, o):
        pltpu.async_copy(text_tab.at[tokbuf[p]], tokrows[p], sem_tok[p])
        pltpu.async_copy(title_tab.at[tidx[p]],
                         outbuf[o].at[:, pl.ds(0, D)], sem_ttl[p])

    def gather_wait(p, o):
        pltpu.make_async_copy(text_tab.at[tokbuf[p]], tokrows[p],
                              sem_tok[p]).wait()
        pltpu.make_async_copy(title_tab.at[tidx[p]],
                              outbuf[o].at[:, pl.ds(0, D)], sem_ttl[p]).wait()

    def out_issue(chunk, o):
        row0 = base + chunk * CHUNK
        pltpu.async_copy(outbuf[o], out_hbm.at[pl.ds(row0, CHUNK)],
                         sem_out[o])

    def out_wait(chunk, o):
        row0 = base + chunk * CHUNK
        pltpu.make_async_copy(outbuf[o], out_hbm.at[pl.ds(row0, CHUNK)],
                              sem_out[o]).wait()

    def counts(p):
        nz = jnp.zeros((LANES,), jnp.float32)
        for t in range(L):
            tv = plsc.load_gather(tokbuf[p], [iota * L + t])
            nz = nz + jnp.where(tv == 0, 1.0, 0.0)
        nzbuf[p][pl.ds(0, LANES)] = nz
        recbuf[p][pl.ds(0, LANES)] = \
            1.0 / jnp.maximum(jnp.float32(L) - nz, 1.0)

    def compute(p, o):
        @plsc.parallel_loop(0, CHUNK, unroll=UNROLL)
        def _(r):
            nzv = jnp.full((LANES,), nzbuf[p][pl.ds(r, LANES)][0],
                           jnp.float32)
            rec = jnp.full((LANES,), recbuf[p][pl.ds(r, LANES)][0],
                           jnp.float32)
            for j in range(D // LANES):
                vals = [tokrows[p][r * L + t, pl.ds(j * LANES, LANES)]
                        for t in range(L)]
                while len(vals) > 1:
                    nxt = [vals[k] + vals[k + 1]
                           for k in range(0, len(vals) - 1, 2)]
                    if len(vals) % 2:
                        nxt.append(vals[-1])
                    vals = nxt
                outbuf[o][r, pl.ds(D + j * LANES, LANES)] = \
                    (vals[0] - nzv * row0v[j]) * rec

    pltpu.sync_copy(tokens_hbm.at[pl.ds(base * L, CL)], tokbuf[0])
    pltpu.sync_copy(titles_hbm.at[pl.ds(base, CHUNK)], tidx[0])
    counts(0)
    gather_issue(0, 0)
    idx_copy(1, 1)

    def outer(i, _):
        for u in range(NOUT):
            c = i * NOUT + u
            p = u % 2
            o = u
            pprev = 1 - p

            gather_wait(p, o)

            @pl.when(c + 1 < N_CHUNKS)
            def _():
                idx_wait(c + 1, pprev)

                @pl.when(c >= 3)
                def _():
                    out_wait(c - 3, (u + 1) % NOUT)

                counts(pprev)
                gather_issue(pprev, (u + 1) % NOUT)

            compute(p, o)
            out_issue(c, o)

            @pl.when(c + 2 < N_CHUNKS)
            def _():
                idx_copy(c + 2, p)
        return 0

    lax.fori_loop(0, N_CHUNKS // NOUT, outer, 0)

    for k in range(NOUT):
        c = N_CHUNKS - NOUT + k
        out_wait(c, c % NOUT)


@functools.partial(jax.jit, static_argnums=())
def _sc_call(titles_i, tokens_i, title_table, text_table):
    mesh = plsc.VectorSubcoreMesh(core_axis_name="c", subcore_axis_name="s")
    return pl.kernel(
        _body,
        out_type=jax.ShapeDtypeStruct((B, D_OUT), jnp.float32),
        mesh=mesh,
        scratch_types=[
            pltpu.VMEM((CL,), jnp.int32),
            pltpu.VMEM((CL,), jnp.int32),
            pltpu.VMEM((CHUNK,), jnp.int32),
            pltpu.VMEM((CHUNK,), jnp.int32),
            pltpu.VMEM((CL, D), jnp.float32),
            pltpu.VMEM((CL, D), jnp.float32),
            pltpu.VMEM((CHUNK, D_OUT), jnp.float32),
            pltpu.VMEM((CHUNK, D_OUT), jnp.float32),
            pltpu.VMEM((CHUNK, D_OUT), jnp.float32),
            pltpu.VMEM((CHUNK, D_OUT), jnp.float32),
            pltpu.VMEM((1, D), jnp.float32),
            pltpu.VMEM((2 * LANES,), jnp.float32),
            pltpu.VMEM((2 * LANES,), jnp.float32),
            pltpu.VMEM((2 * LANES,), jnp.float32),
            pltpu.VMEM((2 * LANES,), jnp.float32),
            pltpu.SemaphoreType.DMA,
            pltpu.SemaphoreType.DMA,
            pltpu.SemaphoreType.DMA,
            pltpu.SemaphoreType.DMA,
            pltpu.SemaphoreType.DMA,
            pltpu.SemaphoreType.DMA,
            pltpu.SemaphoreType.DMA,
            pltpu.SemaphoreType.DMA,
            pltpu.SemaphoreType.DMA,
            pltpu.SemaphoreType.DMA,
        ],
        compiler_params=pltpu.CompilerParams(needs_layout_passes=False),
    )(titles_i, tokens_i, title_table, text_table)


def kernel(titles, tokens, title_table, text_table):
    titles_i = titles.astype(jnp.int32)
    tokens_i = tokens.reshape(-1).astype(jnp.int32)
    return _sc_call(titles_i, tokens_i, title_table, text_table)
